# baseline (speedup 1.0000x reference)
import functools

import jax
import jax.numpy as jnp
from jax.experimental import pallas as pl

G, T, H, E, C = 2, 2048, 2048, 16, 128
TBLK_C = 512


def _first(av, ai, bv, bi):
    return (av > bv) | ((av == bv) & (ai < bi))


def _cex(v, i, islow, j, keepmask):
    pv = jnp.where(islow, jnp.roll(v, -j, 1), jnp.roll(v, j, 1))
    pi = jnp.where(islow, jnp.roll(i, -j, 1), jnp.roll(i, j, 1))
    sf = _first(v, i, pv, pi)
    keep = sf == keepmask
    return jnp.where(keep, v, pv), jnp.where(keep, i, pi)


def _topk_desc(v, i):
    rows, w0 = v.shape
    lane = jax.lax.broadcasted_iota(jnp.int32, (rows, w0), 1)
    want = lane < (w0 // 2)
    islow_by_j = {j: (lane & j) == 0 for j in (1, 2, 4, 8, 16, 32, 64)}
    for k in (2, 4, 8, 16, 32, 64, 128):
        desc = want if k == 128 else want ^ ((lane & k) != 0)
        j = k // 2
        while j >= 1:
            islow = islow_by_j[j]
            v, i = _cex(v, i, islow, j, islow == desc)
            j //= 2
    w = w0
    while w > C:
        h = w // 2
        f = _first(v[:, :h], i[:, :h], v[:, h:w], i[:, h:w])
        v = jnp.where(f, v[:, :h], v[:, h:w])
        i = jnp.where(f, i[:, :h], i[:, h:w])
        desc_h = lane[:, :h] < max(h // 2, C)
        for j in (64, 32, 16, 8, 4, 2, 1):
            islow = islow_by_j[j][:, :h]
            v, i = _cex(v, i, islow, j, islow == desc_h)
        w = h
    return v, i


def _fused_body(x_ref, w_ref, b_ref, ei_ref, eg_ref, z_ref):
    g = pl.program_id(0)
    x = x_ref[0]
    w = w_ref[...]
    b = b_ref[...]
    logits = jax.lax.dot_general(
        w, x, dimension_numbers=(((0,), (1,)), ((), ())),
        preferred_element_type=jnp.float32)
    logits = logits + b.reshape(E, 1)
    m = jnp.max(logits, axis=0, keepdims=True)
    ex = jnp.exp(logits - m)
    s = jnp.sum(ex, axis=0, keepdims=True)
    probs = ex / s
    lse = m + jnp.log(s)
    zpart = jnp.sum(lse * lse).reshape(1, 1)

    @pl.when(g == 0)
    def _():
        z_ref[...] = jnp.zeros_like(z_ref)

    z_ref[...] += zpart

    iota = jax.lax.broadcasted_iota(jnp.int32, (E, T), 1)
    tv, ti = _topk_desc(probs, iota)
    ei_ref[0] = ti
    eg_ref[0] = tv


def _mask_body(ei_ref, eg_ref, disp_ref, comb_ref):
    tb = pl.program_id(1)
    t0 = tb * TBLK_C
    ti = jax.lax.broadcasted_iota(jnp.int32, (TBLK_C, E, C), 0) + t0
    hit = ei_ref[0][None, :, :] == ti
    disp_ref[0] = jnp.where(hit, 1.0, 0.0).astype(jnp.float32)
    comb_ref[0] = jnp.where(hit, eg_ref[0][None, :, :], 0.0).astype(jnp.float32)


@functools.partial(jax.jit, static_argnums=())
def _run(x, w, b):
    ei, eg, zsum = pl.pallas_call(
        _fused_body,
        grid=(G,),
        in_specs=[
            pl.BlockSpec((1, T, H), lambda g: (g, 0, 0)),
            pl.BlockSpec((H, E), lambda g: (0, 0)),
            pl.BlockSpec((1, E), lambda g: (0, 0)),
        ],
        out_specs=[
            pl.BlockSpec((1, E, C), lambda g: (g, 0, 0)),
            pl.BlockSpec((1, E, C), lambda g: (g, 0, 0)),
            pl.BlockSpec((1, 1), lambda g: (0, 0)),
        ],
        out_shape=[
            jax.ShapeDtypeStruct((G, E, C), jnp.int32),
            jax.ShapeDtypeStruct((G, E, C), jnp.float32),
            jax.ShapeDtypeStruct((1, 1), jnp.float32),
        ],
    )(x, w, b.reshape(1, E))

    disp, comb = pl.pallas_call(
        _mask_body,
        grid=(G, T // TBLK_C),
        in_specs=[
            pl.BlockSpec((1, E, C), lambda g, tb: (g, 0, 0)),
            pl.BlockSpec((1, E, C), lambda g, tb: (g, 0, 0)),
        ],
        out_specs=[
            pl.BlockSpec((1, TBLK_C, E, C), lambda g, tb: (g, tb, 0, 0)),
            pl.BlockSpec((1, TBLK_C, E, C), lambda g, tb: (g, tb, 0, 0)),
        ],
        out_shape=[
            jax.ShapeDtypeStruct((G, T, E, C), jnp.float32),
            jax.ShapeDtypeStruct((G, T, E, C), jnp.float32),
        ],
    )(ei, eg)

    z_loss = zsum[0, 0] / (G * T)
    return disp, comb, z_loss


def kernel(inputs, kernel, bias, expert_capacity):
    del expert_capacity
    return _run(inputs, kernel, bias)
